# Initial kernel scaffold; baseline (speedup 1.0000x reference)
import jax
import jax.numpy as jnp
from jax.experimental import pallas as pl


def kernel(x, x_3d, W, feature_map_indexing, atomic_segment_ids, view_segment_ids):
    raise NotImplementedError("write your pallas kernel here")


# speedup vs baseline: 3.5515x; 3.5515x over previous
import jax
import jax.numpy as jnp
from jax import lax
from jax.experimental import pallas as pl
from jax.experimental.pallas import tpu as pltpu
from jax.experimental.pallas import tpu_sc as plsc

N_PIX = 262144
D = 64
M = 524288
N_VIEWS = 131072
N_PTS = 32768

NS = 16
LANES = 16
B = 128
CHUNK = 1024
R = CHUNK // B


def _matmul_body(x_ref, w_ref, o_ref):
    o_ref[...] = jnp.dot(x_ref[...], w_ref[...], preferred_element_type=jnp.float32)


def _matmul(x, w):
    bm = 2048
    return pl.pallas_call(
        _matmul_body,
        grid=(N_PIX // bm,),
        in_specs=[
            pl.BlockSpec((bm, D), lambda i: (i, 0)),
            pl.BlockSpec((D, D), lambda i: (0, 0)),
        ],
        out_specs=pl.BlockSpec((bm, D), lambda i: (i, 0)),
        out_shape=jax.ShapeDtypeStruct((N_PIX, D), jnp.float32),
    )(x, w)


def _fuse_body(x3d_ref, a0_ref, a1_ref, o_ref):
    o_ref[...] = x3d_ref[...] + jnp.concatenate([a0_ref[...], a1_ref[...]], axis=1)


def _fuse(x_3d, accf):
    br = 2048
    nb = N_PTS // br
    return pl.pallas_call(
        _fuse_body,
        grid=(nb,),
        in_specs=[
            pl.BlockSpec((br, D), lambda i: (i, 0)),
            pl.BlockSpec((br, D // 2), lambda i: (i, 0)),
            pl.BlockSpec((br, D // 2), lambda i, nb=nb: (i + nb, 0)),
        ],
        out_specs=pl.BlockSpec((br, D), lambda i: (i, 0)),
        out_shape=jax.ShapeDtypeStruct((N_PTS, D), jnp.float32),
    )(x_3d, accf, accf)


def _sc_body(xm2, fmi2, as2, vs2, out,
             ids_v, bufB, bufC, sbuf, a_v, rows_v, ones_v,
             cnt_a_sp, cnt_v_sp, vs_sp, acc_sp):
    c = lax.axis_index("c")
    s = lax.axis_index("s")

    def _zero_rows(r, _):
        rows_v[r, pl.ds(0, LANES)] = jnp.zeros((LANES,), jnp.float32)
        rows_v[r, pl.ds(LANES, LANES)] = jnp.zeros((LANES,), jnp.float32)
        return 0
    lax.fori_loop(0, CHUNK, _zero_rows, 0)

    def _zero_a(q, _):
        a_v[pl.ds(q * LANES, LANES)] = jnp.zeros((LANES,), jnp.int32)
        return 0
    lax.fori_loop(0, CHUNK // LANES, _zero_a, 0)

    for j in range(B // LANES):
        ones_v[pl.ds(j * LANES, LANES)] = jnp.ones((LANES,), jnp.int32)

    for q in range(8):
        pltpu.sync_copy(a_v, cnt_a_sp.at[pl.ds(s * 8192 + q * CHUNK, CHUNK)])
    for q in range(2):
        pltpu.sync_copy(a_v, cnt_v_sp.at[pl.ds(s * 2048 + q * CHUNK, CHUNK)])
        pltpu.sync_copy(rows_v, acc_sp.at[pl.ds(s * 2048 + q * CHUNK, CHUNK)])
    plsc.subcore_barrier()

    def _count_a(k, _):
        rb = pl.multiple_of(s * 256 + k * R, 8)
        pltpu.sync_copy(as2.at[pl.ds(rb, R)], ids_v)
        for j in range(R):
            pltpu.sync_copy(ones_v, cnt_a_sp.at[ids_v.at[j]], add=True)
        return 0
    lax.fori_loop(0, 32, _count_a, 0)

    def _count_v(k, _):
        rb = pl.multiple_of(s * 64 + k * R, 8)
        pltpu.sync_copy(vs2.at[pl.ds(rb, R)], ids_v)
        for j in range(R):
            pltpu.sync_copy(ones_v, cnt_v_sp.at[ids_v.at[j]], add=True)
            pltpu.sync_copy(ids_v.at[j], vs_sp.at[pl.ds(pl.multiple_of((rb + j) * B, B), B)])
        return 0
    lax.fori_loop(0, 8, _count_v, 0)
    plsc.subcore_barrier()

    def _scale(k, _):
        vb = pl.multiple_of(s * 8192 + k * CHUNK, CHUNK)
        rb = pl.multiple_of(vb // B, 8)
        pltpu.sync_copy(cnt_a_sp.at[pl.ds(vb, CHUNK)], a_v)
        pltpu.sync_copy(vs2.at[pl.ds(rb, R)], ids_v)
        for j in range(R):
            pltpu.sync_copy(cnt_v_sp.at[ids_v.at[j]], bufC.at[j])

        def _t(q, _):
            i = q // 8
            jj = q % 8
            a16 = a_v[pl.ds(q * LANES, LANES)]
            vc16 = bufC[i, pl.ds(jj * LANES, LANES)]
            af = jnp.maximum(a16, 1).astype(jnp.float32)
            vf = vc16.astype(jnp.float32)
            t16 = 1.0 / (af * vf)
            sbuf[i, pl.ds(jj * LANES, LANES)] = plsc.bitcast(t16, jnp.int32)
            return 0
        lax.fori_loop(0, CHUNK // LANES, _t, 0)
        for j in range(R):
            pltpu.sync_copy(sbuf.at[j], cnt_a_sp.at[pl.ds(pl.multiple_of(vb + j * B, B), B)])
        return 0
    lax.fori_loop(0, 8, _scale, 0)
    plsc.subcore_barrier()

    def _main(k, _):
        rb = pl.multiple_of(s * 256 + k * R, 8)
        pltpu.sync_copy(fmi2.at[pl.ds(rb, R)], bufB)
        pltpu.sync_copy(as2.at[pl.ds(rb, R)], ids_v)

        def _gidx(q, _):
            i = q // 8
            jj = q % 8
            v = bufB[i, pl.ds(jj * LANES, LANES)]
            bufB[i, pl.ds(jj * LANES, LANES)] = v * 2 + c
            return 0
        lax.fori_loop(0, CHUNK // LANES, _gidx, 0)

        for j in range(R):
            pltpu.sync_copy(cnt_a_sp.at[ids_v.at[j]], sbuf.at[j])
            pltpu.sync_copy(vs_sp.at[ids_v.at[j]], bufC.at[j])
            pltpu.sync_copy(xm2.at[bufB.at[j]], rows_v.at[pl.ds(j * B, B)])

        def _scalemul(g, _):
            i = g // 8
            jj = g % 8
            s16 = plsc.bitcast(sbuf[i, pl.ds(jj * LANES, LANES)], jnp.float32)
            for u in range(LANES):
                r = g * LANES + u
                sb = jnp.full((LANES,), s16[u], jnp.float32)
                rows_v[r, pl.ds(0, LANES)] = rows_v[r, pl.ds(0, LANES)] * sb
                rows_v[r, pl.ds(LANES, LANES)] = rows_v[r, pl.ds(LANES, LANES)] * sb
            return 0
        lax.fori_loop(0, CHUNK // LANES, _scalemul, 0)

        for j in range(R):
            pltpu.sync_copy(rows_v.at[pl.ds(j * B, B)], acc_sp.at[bufC.at[j]], add=True)
        return 0
    lax.fori_loop(0, 32, _main, 0)
    plsc.subcore_barrier()

    pltpu.sync_copy(acc_sp.at[pl.ds(s * 2048, 2048)],
                    out.at[pl.ds(c * N_PTS + s * 2048, 2048)])


def _sc_pool(xm2, fmi2, as2, vs2):
    mesh = plsc.VectorSubcoreMesh(core_axis_name="c", subcore_axis_name="s")
    f = pl.kernel(
        _sc_body,
        out_type=jax.ShapeDtypeStruct((2 * N_PTS, D // 2), jnp.float32),
        mesh=mesh,
        scratch_types=[
            pltpu.VMEM((R, B), jnp.int32),
            pltpu.VMEM((R, B), jnp.int32),
            pltpu.VMEM((R, B), jnp.int32),
            pltpu.VMEM((R, B), jnp.int32),
            pltpu.VMEM((CHUNK,), jnp.int32),
            pltpu.VMEM((CHUNK, D // 2), jnp.float32),
            pltpu.VMEM((B,), jnp.int32),
            pltpu.VMEM_SHARED((N_VIEWS,), jnp.int32),
            pltpu.VMEM_SHARED((N_PTS,), jnp.int32),
            pltpu.VMEM_SHARED((N_VIEWS,), jnp.int32),
            pltpu.VMEM_SHARED((N_PTS, D // 2), jnp.float32),
        ],
        compiler_params=pltpu.CompilerParams(
            use_tc_tiling_on_sc=False, needs_layout_passes=False),
    )
    return f(xm2, fmi2, as2, vs2)


def kernel(x, x_3d, W, feature_map_indexing, atomic_segment_ids, view_segment_ids):
    x_mod = _matmul(x, W)
    xm2 = x_mod.reshape(2 * N_PIX, D // 2)
    fmi2 = feature_map_indexing.reshape(M // B, B)
    as2 = atomic_segment_ids.reshape(M // B, B)
    vs2 = view_segment_ids.reshape(N_VIEWS // B, B)
    accf = _sc_pool(xm2, fmi2, as2, vs2)
    x_3d_out = _fuse(x_3d, accf)
    return (x_mod, x_3d_out)


# speedup vs baseline: 4.0570x; 1.1423x over previous
import jax
import jax.numpy as jnp
from jax import lax
from jax.experimental import pallas as pl
from jax.experimental.pallas import tpu as pltpu
from jax.experimental.pallas import tpu_sc as plsc

N_PIX = 262144
D = 64
M = 524288
N_VIEWS = 131072
N_PTS = 32768

NS = 16
LANES = 16
B = 128
CHUNK = 1024
R = CHUNK // B


def _matmul_body(x_ref, w_ref, o_ref):
    o_ref[...] = jnp.dot(x_ref[...], w_ref[...], preferred_element_type=jnp.float32)


def _matmul(x, w):
    bm = 2048
    return pl.pallas_call(
        _matmul_body,
        grid=(N_PIX // bm,),
        in_specs=[
            pl.BlockSpec((bm, D), lambda i: (i, 0)),
            pl.BlockSpec((D, D), lambda i: (0, 0)),
        ],
        out_specs=pl.BlockSpec((bm, D), lambda i: (i, 0)),
        out_shape=jax.ShapeDtypeStruct((N_PIX, D), jnp.float32),
    )(x, w)


def _fuse_body(x3d_ref, a0_ref, a1_ref, w_ref, o_ref):
    acc = jnp.concatenate([a0_ref[...], a1_ref[...]], axis=1)
    o_ref[...] = x3d_ref[...] + jnp.dot(acc, w_ref[...],
                                        preferred_element_type=jnp.float32)


def _fuse(x_3d, accf, w):
    br = 2048
    nb = N_PTS // br
    return pl.pallas_call(
        _fuse_body,
        grid=(nb,),
        in_specs=[
            pl.BlockSpec((br, D), lambda i: (i, 0)),
            pl.BlockSpec((br, D // 2), lambda i: (i, 0)),
            pl.BlockSpec((br, D // 2), lambda i, nb=nb: (i + nb, 0)),
            pl.BlockSpec((D, D), lambda i: (0, 0)),
        ],
        out_specs=pl.BlockSpec((br, D), lambda i: (i, 0)),
        out_shape=jax.ShapeDtypeStruct((N_PTS, D), jnp.float32),
    )(x_3d, accf, accf, w)


def _sc_body(xm2, fmi2, as2, vs2, out,
             ids_v, bufB, bufC, sbuf, a_v, rows_v, ones_v,
             cnt_a_sp, cnt_v_sp, vs_sp, acc_sp):
    c = lax.axis_index("c")
    s = lax.axis_index("s")

    def _zero_rows(r, _):
        rows_v[r, pl.ds(0, LANES)] = jnp.zeros((LANES,), jnp.float32)
        rows_v[r, pl.ds(LANES, LANES)] = jnp.zeros((LANES,), jnp.float32)
        return 0
    lax.fori_loop(0, CHUNK, _zero_rows, 0)

    def _zero_a(q, _):
        a_v[pl.ds(q * LANES, LANES)] = jnp.zeros((LANES,), jnp.int32)
        return 0
    lax.fori_loop(0, CHUNK // LANES, _zero_a, 0)

    for j in range(B // LANES):
        ones_v[pl.ds(j * LANES, LANES)] = jnp.ones((LANES,), jnp.int32)

    for q in range(8):
        pltpu.sync_copy(a_v, cnt_a_sp.at[pl.ds(s * 8192 + q * CHUNK, CHUNK)])
    for q in range(2):
        pltpu.sync_copy(a_v, cnt_v_sp.at[pl.ds(s * 2048 + q * CHUNK, CHUNK)])
        pltpu.sync_copy(rows_v, acc_sp.at[pl.ds(s * 2048 + q * CHUNK, CHUNK)])
    plsc.subcore_barrier()

    def _count_a(k, _):
        rb = pl.multiple_of(s * 256 + k * R, 8)
        pltpu.sync_copy(as2.at[pl.ds(rb, R)], ids_v)
        for j in range(R):
            pltpu.sync_copy(ones_v, cnt_a_sp.at[ids_v.at[j]], add=True)
        return 0
    lax.fori_loop(0, 32, _count_a, 0)

    def _count_v(k, _):
        rb = pl.multiple_of(s * 64 + k * R, 8)
        pltpu.sync_copy(vs2.at[pl.ds(rb, R)], ids_v)
        for j in range(R):
            pltpu.sync_copy(ones_v, cnt_v_sp.at[ids_v.at[j]], add=True)
            pltpu.sync_copy(ids_v.at[j], vs_sp.at[pl.ds(pl.multiple_of((rb + j) * B, B), B)])
        return 0
    lax.fori_loop(0, 8, _count_v, 0)
    plsc.subcore_barrier()

    def _scale(k, _):
        vb = pl.multiple_of(s * 8192 + k * CHUNK, CHUNK)
        rb = pl.multiple_of(vb // B, 8)
        pltpu.sync_copy(cnt_a_sp.at[pl.ds(vb, CHUNK)], a_v)
        pltpu.sync_copy(vs2.at[pl.ds(rb, R)], ids_v)
        for j in range(R):
            pltpu.sync_copy(cnt_v_sp.at[ids_v.at[j]], bufC.at[j])

        def _t(q, _):
            i = q // 8
            jj = q % 8
            a16 = a_v[pl.ds(q * LANES, LANES)]
            vc16 = bufC[i, pl.ds(jj * LANES, LANES)]
            af = jnp.maximum(a16, 1).astype(jnp.float32)
            vf = vc16.astype(jnp.float32)
            t16 = 1.0 / (af * vf)
            sbuf[i, pl.ds(jj * LANES, LANES)] = plsc.bitcast(t16, jnp.int32)
            return 0
        lax.fori_loop(0, CHUNK // LANES, _t, 0)
        for j in range(R):
            pltpu.sync_copy(sbuf.at[j], cnt_a_sp.at[pl.ds(pl.multiple_of(vb + j * B, B), B)])
        return 0
    lax.fori_loop(0, 8, _scale, 0)
    plsc.subcore_barrier()

    def _main(k, _):
        rb = pl.multiple_of(s * 256 + k * R, 8)
        pltpu.sync_copy(fmi2.at[pl.ds(rb, R)], bufB)
        pltpu.sync_copy(as2.at[pl.ds(rb, R)], ids_v)

        def _gidx(q, _):
            i = q // 8
            jj = q % 8
            v = bufB[i, pl.ds(jj * LANES, LANES)]
            bufB[i, pl.ds(jj * LANES, LANES)] = v * 2 + c
            return 0
        lax.fori_loop(0, CHUNK // LANES, _gidx, 0)

        for j in range(R):
            pltpu.sync_copy(cnt_a_sp.at[ids_v.at[j]], sbuf.at[j])
            pltpu.sync_copy(vs_sp.at[ids_v.at[j]], bufC.at[j])
            pltpu.sync_copy(xm2.at[bufB.at[j]], rows_v.at[pl.ds(j * B, B)])

        def _scalemul(g, _):
            i = g // 8
            jj = g % 8
            s16 = plsc.bitcast(sbuf[i, pl.ds(jj * LANES, LANES)], jnp.float32)
            for u in range(LANES):
                r = g * LANES + u
                sb = jnp.full((LANES,), s16[u], jnp.float32)
                rows_v[r, pl.ds(0, LANES)] = rows_v[r, pl.ds(0, LANES)] * sb
                rows_v[r, pl.ds(LANES, LANES)] = rows_v[r, pl.ds(LANES, LANES)] * sb
            return 0
        lax.fori_loop(0, CHUNK // LANES, _scalemul, 0)

        for j in range(R):
            pltpu.sync_copy(rows_v.at[pl.ds(j * B, B)], acc_sp.at[bufC.at[j]], add=True)
        return 0
    lax.fori_loop(0, 32, _main, 0)
    plsc.subcore_barrier()

    pltpu.sync_copy(acc_sp.at[pl.ds(s * 2048, 2048)],
                    out.at[pl.ds(c * N_PTS + s * 2048, 2048)])


def _sc_pool(xm2, fmi2, as2, vs2):
    mesh = plsc.VectorSubcoreMesh(core_axis_name="c", subcore_axis_name="s")
    f = pl.kernel(
        _sc_body,
        out_type=jax.ShapeDtypeStruct((2 * N_PTS, D // 2), jnp.float32),
        mesh=mesh,
        scratch_types=[
            pltpu.VMEM((R, B), jnp.int32),
            pltpu.VMEM((R, B), jnp.int32),
            pltpu.VMEM((R, B), jnp.int32),
            pltpu.VMEM((R, B), jnp.int32),
            pltpu.VMEM((CHUNK,), jnp.int32),
            pltpu.VMEM((CHUNK, D // 2), jnp.float32),
            pltpu.VMEM((B,), jnp.int32),
            pltpu.VMEM_SHARED((N_VIEWS,), jnp.int32),
            pltpu.VMEM_SHARED((N_PTS,), jnp.int32),
            pltpu.VMEM_SHARED((N_VIEWS,), jnp.int32),
            pltpu.VMEM_SHARED((N_PTS, D // 2), jnp.float32),
        ],
        compiler_params=pltpu.CompilerParams(
            use_tc_tiling_on_sc=False, needs_layout_passes=False),
    )
    return f(xm2, fmi2, as2, vs2)


def kernel(x, x_3d, W, feature_map_indexing, atomic_segment_ids, view_segment_ids):
    x2 = x.reshape(2 * N_PIX, D // 2)
    fmi2 = feature_map_indexing.reshape(M // B, B)
    as2 = atomic_segment_ids.reshape(M // B, B)
    vs2 = view_segment_ids.reshape(N_VIEWS // B, B)
    accf = _sc_pool(x2, fmi2, as2, vs2)
    x_mod = _matmul(x, W)
    x_3d_out = _fuse(x_3d, accf, W)
    return (x_mod, x_3d_out)


# speedup vs baseline: 4.1777x; 1.0298x over previous
import jax
import jax.numpy as jnp
from jax import lax
from jax.experimental import pallas as pl
from jax.experimental.pallas import tpu as pltpu
from jax.experimental.pallas import tpu_sc as plsc

N_PIX = 262144
D = 64
M = 524288
N_VIEWS = 131072
N_PTS = 32768

NS = 16
LANES = 16
B = 128
CHUNK = 1024
R = CHUNK // B


def _matmul_body(x_ref, w_ref, o_ref):
    o_ref[...] = lax.dot_general(
        w_ref[...], x_ref[...], (((0,), (1,)), ((), ())),
        preferred_element_type=jnp.float32)


def _matmul(x, w):
    bm = 2048
    out_t = pl.pallas_call(
        _matmul_body,
        grid=(N_PIX // bm,),
        in_specs=[
            pl.BlockSpec((bm, D), lambda i: (i, 0)),
            pl.BlockSpec((D, D), lambda i: (0, 0)),
        ],
        out_specs=pl.BlockSpec((D, bm), lambda i: (0, i)),
        out_shape=jax.ShapeDtypeStruct((D, N_PIX), jnp.float32),
    )(x, w)
    return out_t.T


def _fuse_body(x3d_ref, a0_ref, a1_ref, w_ref, o_ref):
    acc = jnp.concatenate([a0_ref[...], a1_ref[...]], axis=1)
    o_ref[...] = x3d_ref[...] + jnp.dot(acc, w_ref[...],
                                        preferred_element_type=jnp.float32)


def _fuse(x_3d, accf, w):
    br = 2048
    nb = N_PTS // br
    return pl.pallas_call(
        _fuse_body,
        grid=(nb,),
        in_specs=[
            pl.BlockSpec((br, D), lambda i: (i, 0)),
            pl.BlockSpec((br, D // 2), lambda i: (i, 0)),
            pl.BlockSpec((br, D // 2), lambda i, nb=nb: (i + nb, 0)),
            pl.BlockSpec((D, D), lambda i: (0, 0)),
        ],
        out_specs=pl.BlockSpec((br, D), lambda i: (i, 0)),
        out_shape=jax.ShapeDtypeStruct((N_PTS, D), jnp.float32),
    )(x_3d, accf, accf, w)


def _sc_body(xm2, fmi2, as2, vs2, out,
             ids_v, bufB, bufC, sbuf, a_v, rows_v, ones_v,
             cnt_a_sp, cnt_v_sp, vs_sp, acc_sp):
    c = lax.axis_index("c")
    s = lax.axis_index("s")

    def _zero_rows(r, _):
        rows_v[r, pl.ds(0, LANES)] = jnp.zeros((LANES,), jnp.float32)
        rows_v[r, pl.ds(LANES, LANES)] = jnp.zeros((LANES,), jnp.float32)
        return 0
    lax.fori_loop(0, CHUNK, _zero_rows, 0)

    def _zero_a(q, _):
        a_v[pl.ds(q * LANES, LANES)] = jnp.zeros((LANES,), jnp.int32)
        return 0
    lax.fori_loop(0, CHUNK // LANES, _zero_a, 0)

    for j in range(B // LANES):
        ones_v[pl.ds(j * LANES, LANES)] = jnp.ones((LANES,), jnp.int32)

    for q in range(8):
        pltpu.sync_copy(a_v, cnt_a_sp.at[pl.ds(s * 8192 + q * CHUNK, CHUNK)])
    for q in range(2):
        pltpu.sync_copy(a_v, cnt_v_sp.at[pl.ds(s * 2048 + q * CHUNK, CHUNK)])
        pltpu.sync_copy(rows_v, acc_sp.at[pl.ds(s * 2048 + q * CHUNK, CHUNK)])
    plsc.subcore_barrier()

    def _count_a(k, _):
        rb = pl.multiple_of(s * 256 + k * R, 8)
        pltpu.sync_copy(as2.at[pl.ds(rb, R)], ids_v)
        for j in range(R):
            pltpu.sync_copy(ones_v, cnt_a_sp.at[ids_v.at[j]], add=True)
        return 0
    lax.fori_loop(0, 32, _count_a, 0)

    def _count_v(k, _):
        rb = pl.multiple_of(s * 64 + k * R, 8)
        pltpu.sync_copy(vs2.at[pl.ds(rb, R)], ids_v)
        for j in range(R):
            pltpu.sync_copy(ones_v, cnt_v_sp.at[ids_v.at[j]], add=True)
            pltpu.sync_copy(ids_v.at[j], vs_sp.at[pl.ds(pl.multiple_of((rb + j) * B, B), B)])
        return 0
    lax.fori_loop(0, 8, _count_v, 0)
    plsc.subcore_barrier()

    def _scale(k, _):
        vb = pl.multiple_of(s * 8192 + k * CHUNK, CHUNK)
        rb = pl.multiple_of(vb // B, 8)
        pltpu.sync_copy(cnt_a_sp.at[pl.ds(vb, CHUNK)], a_v)
        pltpu.sync_copy(vs2.at[pl.ds(rb, R)], ids_v)
        for j in range(R):
            pltpu.sync_copy(cnt_v_sp.at[ids_v.at[j]], bufC.at[j])

        def _t(q, _):
            i = q // 8
            jj = q % 8
            a16 = a_v[pl.ds(q * LANES, LANES)]
            vc16 = bufC[i, pl.ds(jj * LANES, LANES)]
            af = jnp.maximum(a16, 1).astype(jnp.float32)
            vf = vc16.astype(jnp.float32)
            t16 = 1.0 / (af * vf)
            sbuf[i, pl.ds(jj * LANES, LANES)] = plsc.bitcast(t16, jnp.int32)
            return 0
        lax.fori_loop(0, CHUNK // LANES, _t, 0)
        for j in range(R):
            pltpu.sync_copy(sbuf.at[j], cnt_a_sp.at[pl.ds(pl.multiple_of(vb + j * B, B), B)])
        return 0
    lax.fori_loop(0, 8, _scale, 0)
    plsc.subcore_barrier()

    def _main(k, _):
        rb = pl.multiple_of(s * 256 + k * R, 8)
        pltpu.sync_copy(fmi2.at[pl.ds(rb, R)], bufB)
        pltpu.sync_copy(as2.at[pl.ds(rb, R)], ids_v)

        def _gidx(q, _):
            i = q // 8
            jj = q % 8
            v = bufB[i, pl.ds(jj * LANES, LANES)]
            bufB[i, pl.ds(jj * LANES, LANES)] = v * 2 + c
            return 0
        lax.fori_loop(0, CHUNK // LANES, _gidx, 0)

        for j in range(R):
            pltpu.sync_copy(cnt_a_sp.at[ids_v.at[j]], sbuf.at[j])
            pltpu.sync_copy(vs_sp.at[ids_v.at[j]], bufC.at[j])
            pltpu.sync_copy(xm2.at[bufB.at[j]], rows_v.at[pl.ds(j * B, B)])

        def _scalemul(g, _):
            i = g // 8
            jj = g % 8
            s16 = plsc.bitcast(sbuf[i, pl.ds(jj * LANES, LANES)], jnp.float32)
            for u in range(LANES):
                r = g * LANES + u
                sb = jnp.full((LANES,), s16[u], jnp.float32)
                rows_v[r, pl.ds(0, LANES)] = rows_v[r, pl.ds(0, LANES)] * sb
                rows_v[r, pl.ds(LANES, LANES)] = rows_v[r, pl.ds(LANES, LANES)] * sb
            return 0
        lax.fori_loop(0, CHUNK // LANES, _scalemul, 0)

        for j in range(R):
            pltpu.sync_copy(rows_v.at[pl.ds(j * B, B)], acc_sp.at[bufC.at[j]], add=True)
        return 0
    lax.fori_loop(0, 32, _main, 0)
    plsc.subcore_barrier()

    pltpu.sync_copy(acc_sp.at[pl.ds(s * 2048, 2048)],
                    out.at[pl.ds(c * N_PTS + s * 2048, 2048)])


def _sc_pool(xm2, fmi2, as2, vs2):
    mesh = plsc.VectorSubcoreMesh(core_axis_name="c", subcore_axis_name="s")
    f = pl.kernel(
        _sc_body,
        out_type=jax.ShapeDtypeStruct((2 * N_PTS, D // 2), jnp.float32),
        mesh=mesh,
        scratch_types=[
            pltpu.VMEM((R, B), jnp.int32),
            pltpu.VMEM((R, B), jnp.int32),
            pltpu.VMEM((R, B), jnp.int32),
            pltpu.VMEM((R, B), jnp.int32),
            pltpu.VMEM((CHUNK,), jnp.int32),
            pltpu.VMEM((CHUNK, D // 2), jnp.float32),
            pltpu.VMEM((B,), jnp.int32),
            pltpu.VMEM_SHARED((N_VIEWS,), jnp.int32),
            pltpu.VMEM_SHARED((N_PTS,), jnp.int32),
            pltpu.VMEM_SHARED((N_VIEWS,), jnp.int32),
            pltpu.VMEM_SHARED((N_PTS, D // 2), jnp.float32),
        ],
        compiler_params=pltpu.CompilerParams(
            use_tc_tiling_on_sc=False, needs_layout_passes=False),
    )
    return f(xm2, fmi2, as2, vs2)


def kernel(x, x_3d, W, feature_map_indexing, atomic_segment_ids, view_segment_ids):
    x2 = x.reshape(2 * N_PIX, D // 2)
    fmi2 = feature_map_indexing.reshape(M // B, B)
    as2 = atomic_segment_ids.reshape(M // B, B)
    vs2 = view_segment_ids.reshape(N_VIEWS // B, B)
    accf = _sc_pool(x2, fmi2, as2, vs2)
    x_mod = _matmul(x, W)
    x_3d_out = _fuse(x_3d, accf, W)
    return (x_mod, x_3d_out)


# speedup vs baseline: 5.2239x; 1.2504x over previous
import jax
import jax.numpy as jnp
from jax import lax
from jax.experimental import pallas as pl
from jax.experimental.pallas import tpu as pltpu
from jax.experimental.pallas import tpu_sc as plsc

N_PIX = 262144
D = 64
M = 524288
N_VIEWS = 131072
N_PTS = 32768

NS = 16
LANES = 16
B = 128
CHUNK = 1024
R = CHUNK // B


def _matmul_body(x_ref, w_ref, o_ref):
    o_ref[...] = lax.dot_general(
        w_ref[...], x_ref[...], (((0,), (1,)), ((), ())),
        preferred_element_type=jnp.float32)


def _matmul(x, w):
    bm = 2048
    out_t = pl.pallas_call(
        _matmul_body,
        grid=(N_PIX // bm,),
        in_specs=[
            pl.BlockSpec((bm, D), lambda i: (i, 0)),
            pl.BlockSpec((D, D), lambda i: (0, 0)),
        ],
        out_specs=pl.BlockSpec((D, bm), lambda i: (0, i)),
        out_shape=jax.ShapeDtypeStruct((D, N_PIX), jnp.float32),
    )(x, w)
    return out_t.T


def _fuse_body(x3d_ref, a0_ref, a1_ref, w_ref, o_ref):
    acc = jnp.concatenate([a0_ref[...], a1_ref[...]], axis=1)
    o_ref[...] = x3d_ref[...] + jnp.dot(acc, w_ref[...],
                                        preferred_element_type=jnp.float32)


def _fuse(x_3d, accf, w):
    br = 2048
    nb = N_PTS // br
    return pl.pallas_call(
        _fuse_body,
        grid=(nb,),
        in_specs=[
            pl.BlockSpec((br, D), lambda i: (i, 0)),
            pl.BlockSpec((br, D // 2), lambda i: (i, 0)),
            pl.BlockSpec((br, D // 2), lambda i, nb=nb: (i + nb, 0)),
            pl.BlockSpec((D, D), lambda i: (0, 0)),
        ],
        out_specs=pl.BlockSpec((br, D), lambda i: (i, 0)),
        out_shape=jax.ShapeDtypeStruct((N_PTS, D), jnp.float32),
    )(x_3d, accf, accf, w)


def _sc_body(xm2, fmi2, as2, vs2, out,
             ids_v, bufB, bufC, sbuf, a_v, rows_v, ones_v,
             cnt_a_sp, cnt_v_sp, vs_sp, acc_sp, sem_g):
    c = lax.axis_index("c")
    s = lax.axis_index("s")

    def _zero_rows(r, _):
        rows_v[r, pl.ds(0, LANES)] = jnp.zeros((LANES,), jnp.float32)
        rows_v[r, pl.ds(LANES, LANES)] = jnp.zeros((LANES,), jnp.float32)
        return 0
    lax.fori_loop(0, CHUNK, _zero_rows, 0)

    def _zero_a(q, _):
        a_v[pl.ds(q * LANES, LANES)] = jnp.zeros((LANES,), jnp.int32)
        return 0
    lax.fori_loop(0, CHUNK // LANES, _zero_a, 0)

    for j in range(B // LANES):
        ones_v[pl.ds(j * LANES, LANES)] = jnp.ones((LANES,), jnp.int32)

    for q in range(8):
        pltpu.sync_copy(a_v, cnt_a_sp.at[pl.ds(s * 8192 + q * CHUNK, CHUNK)])
    for q in range(2):
        pltpu.sync_copy(a_v, cnt_v_sp.at[pl.ds(s * 2048 + q * CHUNK, CHUNK)])
        pltpu.sync_copy(rows_v, acc_sp.at[pl.ds(s * 2048 + q * CHUNK, CHUNK)])
    plsc.subcore_barrier()

    def _count_a(k, _):
        rb = pl.multiple_of(s * 256 + k * R, 8)
        pltpu.sync_copy(as2.at[pl.ds(rb, R)], ids_v)
        for j in range(R):
            pltpu.sync_copy(ones_v, cnt_a_sp.at[ids_v.at[j]], add=True)
        return 0
    lax.fori_loop(0, 32, _count_a, 0)

    def _count_v(k, _):
        rb = pl.multiple_of(s * 64 + k * R, 8)
        pltpu.sync_copy(vs2.at[pl.ds(rb, R)], ids_v)
        for j in range(R):
            pltpu.sync_copy(ones_v, cnt_v_sp.at[ids_v.at[j]], add=True)
            pltpu.sync_copy(ids_v.at[j], vs_sp.at[pl.ds(pl.multiple_of((rb + j) * B, B), B)])
        return 0
    lax.fori_loop(0, 8, _count_v, 0)
    plsc.subcore_barrier()

    def _scale(k, _):
        vb = pl.multiple_of(s * 8192 + k * CHUNK, CHUNK)
        rb = pl.multiple_of(vb // B, 8)
        pltpu.sync_copy(cnt_a_sp.at[pl.ds(vb, CHUNK)], a_v)
        pltpu.sync_copy(vs2.at[pl.ds(rb, R)], ids_v)
        for j in range(R):
            pltpu.sync_copy(cnt_v_sp.at[ids_v.at[j]], bufC.at[j])

        def _t(q, _):
            i = q // 8
            jj = q % 8
            a16 = a_v[pl.ds(q * LANES, LANES)]
            vc16 = bufC[i, pl.ds(jj * LANES, LANES)]
            af = jnp.maximum(a16, 1).astype(jnp.float32)
            vf = vc16.astype(jnp.float32)
            t16 = 1.0 / (af * vf)
            sbuf[i, pl.ds(jj * LANES, LANES)] = plsc.bitcast(t16, jnp.int32)
            return 0
        lax.fori_loop(0, CHUNK // LANES, _t, 0)
        for j in range(R):
            pltpu.sync_copy(sbuf.at[j], cnt_a_sp.at[pl.ds(pl.multiple_of(vb + j * B, B), B)])
        return 0
    lax.fori_loop(0, 8, _scale, 0)
    plsc.subcore_barrier()

    RH = R // 2

    def _main(h, _):
        half = h % 2
        prev = (h - 1) % 2

        @pl.when(h < 64)
        def _fire():
            rb = pl.multiple_of(s * 256 + h * RH, 4)
            pltpu.sync_copy(fmi2.at[pl.ds(rb, RH)], bufB.at[pl.ds(half * RH, RH)])
            pltpu.sync_copy(as2.at[pl.ds(rb, RH)], ids_v.at[pl.ds(half * RH, RH)])

            def _gidx(q, _):
                i = half * RH + q // 8
                jj = q % 8
                v = bufB[i, pl.ds(jj * LANES, LANES)]
                bufB[i, pl.ds(jj * LANES, LANES)] = v * 2 + c
                return 0
            lax.fori_loop(0, (CHUNK // 2) // LANES, _gidx, 0)

            for j in range(RH):
                jh = half * RH + j
                pltpu.sync_copy(cnt_a_sp.at[ids_v.at[jh]], sbuf.at[jh])
                pltpu.sync_copy(vs_sp.at[ids_v.at[jh]], bufC.at[jh])
                pltpu.async_copy(xm2.at[bufB.at[jh]],
                                 rows_v.at[pl.ds(jh * B, B)],
                                 sem_g.at[half])

        @pl.when(h > 0)
        def _compute():
            base = prev * (CHUNK // 2)
            pltpu.make_async_copy(
                xm2.at[pl.ds(0, CHUNK // 2)],
                rows_v.at[pl.ds(base, CHUNK // 2)],
                sem_g.at[prev]).wait()

            def _scalemul(g, _):
                i = prev * RH + g // 8
                jj = g % 8
                s16 = plsc.bitcast(sbuf[i, pl.ds(jj * LANES, LANES)], jnp.float32)
                for u in range(LANES):
                    r = base + g * LANES + u
                    sb = jnp.full((LANES,), s16[u], jnp.float32)
                    rows_v[r, pl.ds(0, LANES)] = rows_v[r, pl.ds(0, LANES)] * sb
                    rows_v[r, pl.ds(LANES, LANES)] = rows_v[r, pl.ds(LANES, LANES)] * sb
                return 0
            lax.fori_loop(0, (CHUNK // 2) // LANES, _scalemul, 0)

            for j in range(RH):
                jh = prev * RH + j
                pltpu.sync_copy(rows_v.at[pl.ds(jh * B, B)],
                                acc_sp.at[bufC.at[jh]], add=True)
        return 0
    lax.fori_loop(0, 65, _main, 0)
    plsc.subcore_barrier()

    pltpu.sync_copy(acc_sp.at[pl.ds(s * 2048, 2048)],
                    out.at[pl.ds(c * N_PTS + s * 2048, 2048)])


def _sc_pool(xm2, fmi2, as2, vs2):
    mesh = plsc.VectorSubcoreMesh(core_axis_name="c", subcore_axis_name="s")
    f = pl.kernel(
        _sc_body,
        out_type=jax.ShapeDtypeStruct((2 * N_PTS, D // 2), jnp.float32),
        mesh=mesh,
        scratch_types=[
            pltpu.VMEM((R, B), jnp.int32),
            pltpu.VMEM((R, B), jnp.int32),
            pltpu.VMEM((R, B), jnp.int32),
            pltpu.VMEM((R, B), jnp.int32),
            pltpu.VMEM((CHUNK,), jnp.int32),
            pltpu.VMEM((CHUNK, D // 2), jnp.float32),
            pltpu.VMEM((B,), jnp.int32),
            pltpu.VMEM_SHARED((N_VIEWS,), jnp.int32),
            pltpu.VMEM_SHARED((N_PTS,), jnp.int32),
            pltpu.VMEM_SHARED((N_VIEWS,), jnp.int32),
            pltpu.VMEM_SHARED((N_PTS, D // 2), jnp.float32),
            pltpu.SemaphoreType.DMA((2,)),
        ],
        compiler_params=pltpu.CompilerParams(
            use_tc_tiling_on_sc=False, needs_layout_passes=False),
    )
    return f(xm2, fmi2, as2, vs2)


def kernel(x, x_3d, W, feature_map_indexing, atomic_segment_ids, view_segment_ids):
    x2 = x.reshape(2 * N_PIX, D // 2)
    fmi2 = feature_map_indexing.reshape(M // B, B)
    as2 = atomic_segment_ids.reshape(M // B, B)
    vs2 = view_segment_ids.reshape(N_VIEWS // B, B)
    accf = _sc_pool(x2, fmi2, as2, vs2)
    x_mod = _matmul(x, W)
    x_3d_out = _fuse(x_3d, accf, W)
    return (x_mod, x_3d_out)
